# baseline (speedup 1.0000x reference)
import functools

import jax
import jax.numpy as jnp
from jax import lax
from jax.experimental import pallas as pl
from jax.experimental.pallas import tpu as pltpu
from jax.experimental.pallas import tpu_sc as plsc

D = 768
E = 64
T = 2048
TOPK = 2
TR = 64
MAX_TILES = T * TOPK // TR + E
T_PAD = MAX_TILES * TR
NW = 32
_BLK = 128


def _route_plan_body(x_ref, wr_ref, br_ref,
                     wa_ref, wb_ref, da_ref, db_ref, te_ref, nt_ref):
    f32 = jnp.float32
    logits = jnp.dot(x_ref[...], wr_ref[...], preferred_element_type=f32)
    logits = logits + br_ref[...]
    iota = lax.broadcasted_iota(jnp.int32, (T, E), 1)
    m1 = jnp.max(logits, axis=-1, keepdims=True)
    i1 = jnp.min(jnp.where(logits == m1, iota, E), axis=-1, keepdims=True)
    masked = jnp.where(iota == i1, -jnp.inf, logits)
    m2 = jnp.max(masked, axis=-1, keepdims=True)
    i2 = jnp.min(jnp.where(masked == m2, iota, E), axis=-1, keepdims=True)
    e2 = jnp.exp(m2 - m1)
    s = 1.0 + e2
    wa_ref[...] = 1.0 / s
    wb_ref[...] = e2 / s

    ohA = (iota == i1).astype(f32)
    ohB = (iota == i2).astype(f32)
    counts = jnp.sum(ohA + ohB, axis=0, keepdims=True)
    tiles = jnp.floor((counts + (TR - 1)) * (1.0 / TR))
    le = (lax.broadcasted_iota(jnp.int32, (E, E), 0)
          <= lax.broadcasted_iota(jnp.int32, (E, E), 1)).astype(f32)
    cum_tiles = jnp.dot(tiles, le, preferred_element_type=f32)
    padded_off = (cum_tiles - tiles) * TR
    nt = jnp.max(cum_tiles, axis=-1, keepdims=True)
    jcol = lax.broadcasted_iota(jnp.int32, (MAX_TILES, E), 0).astype(f32)
    jcl = jnp.minimum(jcol, nt - 1.0)
    te = jnp.sum(jnp.where(cum_tiles <= jcl, 1.0, 0.0), axis=-1, keepdims=True)
    te_ref[...] = te.astype(jnp.int32)
    nt_ref[...] = nt.astype(jnp.int32)

    bi = lax.broadcasted_iota(jnp.int32, (_BLK, _BLK), 0)
    bj = lax.broadcasted_iota(jnp.int32, (_BLK, _BLK), 1)
    lx = (bj < bi).astype(f32)
    carry = jnp.zeros((1, E), f32)
    for blk in range(T // _BLK):
        lo, hi = blk * _BLK, (blk + 1) * _BLK
        oa = ohA[lo:hi, :]
        ob = ohB[lo:hi, :]
        ra = jnp.dot(lx, oa, preferred_element_type=f32) + carry
        carry = carry + jnp.sum(oa, axis=0, keepdims=True)
        rb = jnp.dot(lx, ob, preferred_element_type=f32) + carry
        carry = carry + jnp.sum(ob, axis=0, keepdims=True)
        da = jnp.sum((ra + padded_off) * oa, axis=-1, keepdims=True)
        db = jnp.sum((rb + padded_off) * ob, axis=-1, keepdims=True)
        da_ref[lo:hi, :] = da.astype(jnp.int32)
        db_ref[lo:hi, :] = db.astype(jnp.int32)


def _route_plan(x, Wr, br, interpret=False):
    return pl.pallas_call(
        _route_plan_body,
        out_shape=[
            jax.ShapeDtypeStruct((T, 1), jnp.float32),
            jax.ShapeDtypeStruct((T, 1), jnp.float32),
            jax.ShapeDtypeStruct((T, 1), jnp.int32),
            jax.ShapeDtypeStruct((T, 1), jnp.int32),
            jax.ShapeDtypeStruct((MAX_TILES, 1), jnp.int32),
            jax.ShapeDtypeStruct((1, 1), jnp.int32),
        ],
        interpret=interpret,
    )(x, Wr, br.reshape(1, E))


_SC_MESH = dict(core_axis_name="c", subcore_axis_name="s")
_S_ROWS = T // NW


def _scatter_sc(x, da, db):
    @functools.partial(
        pl.kernel,
        mesh=plsc.VectorSubcoreMesh(**_SC_MESH),
        out_type=jax.ShapeDtypeStruct((T_PAD, D), jnp.float32),
        scratch_types=[
            pltpu.VMEM((_S_ROWS,), jnp.int32),
            pltpu.VMEM((_S_ROWS,), jnp.int32),
            pltpu.VMEM((_S_ROWS, D), jnp.float32),
            pltpu.SemaphoreType.DMA,
            pltpu.SemaphoreType.DMA,
        ],
    )
    def k(x_hbm, da_hbm, db_hbm, xs_hbm, ia, ib, rows_v, s1, s2):
        wid = lax.axis_index("s") * 2 + lax.axis_index("c")
        base = wid * _S_ROWS
        pltpu.sync_copy(x_hbm.at[pl.ds(base, _S_ROWS)], rows_v)
        pltpu.sync_copy(da_hbm.at[pl.ds(base, _S_ROWS)], ia)
        pltpu.sync_copy(db_hbm.at[pl.ds(base, _S_ROWS)], ib)
        ca = pltpu.async_copy(rows_v, xs_hbm.at[ia], s1)
        cb = pltpu.async_copy(rows_v, xs_hbm.at[ib], s2)
        ca.wait()
        cb.wait()

    return k(x, da, db)


_NSPLIT = 4


def _mlp_body(te_ref, nt_ref, xs_ref, *refs):
    w1_refs = refs[:_NSPLIT]
    b1_ref = refs[_NSPLIT]
    w2_refs = refs[_NSPLIT + 1:2 * _NSPLIT + 1]
    b2_ref = refs[2 * _NSPLIT + 1]
    ys_ref = refs[2 * _NSPLIT + 2]
    j = pl.program_id(0)
    hc = 4 * D // _NSPLIT

    @pl.when(j < nt_ref[0])
    def _():
        xt = xs_ref[...]
        y = b2_ref[0, 0, :]
        for c in range(_NSPLIT):
            h = jnp.dot(xt, w1_refs[c][0], preferred_element_type=jnp.float32)
            h = jnp.maximum(h + b1_ref[0, 0, c * hc:(c + 1) * hc], 0.0)
            y = y + jnp.dot(h, w2_refs[c][0],
                            preferred_element_type=jnp.float32)
        ys_ref[...] = y


def _mlp(tile_expert, n_tiles, xs, W1, b1, W2, b2, interpret=False):
    hc = 4 * D // _NSPLIT
    grid_spec = pltpu.PrefetchScalarGridSpec(
        num_scalar_prefetch=2,
        grid=(MAX_TILES,),
        in_specs=(
            [pl.BlockSpec((TR, D), lambda j, te, nt: (j, 0))]
            + [pl.BlockSpec((1, D, hc),
                            lambda j, te, nt, c=c: (te[j], 0, c))
               for c in range(_NSPLIT)]
            + [pl.BlockSpec((1, 1, 4 * D), lambda j, te, nt: (te[j], 0, 0))]
            + [pl.BlockSpec((1, hc, D),
                            lambda j, te, nt, c=c: (te[j], c, 0))
               for c in range(_NSPLIT)]
            + [pl.BlockSpec((1, 1, D), lambda j, te, nt: (te[j], 0, 0))]
        ),
        out_specs=pl.BlockSpec((TR, D), lambda j, te, nt: (j, 0)),
    )
    return pl.pallas_call(
        _mlp_body,
        grid_spec=grid_spec,
        out_shape=jax.ShapeDtypeStruct((T_PAD, D), jnp.float32),
        compiler_params=pltpu.CompilerParams(
            dimension_semantics=("arbitrary",),
            vmem_limit_bytes=120 * 1024 * 1024,
        ),
        interpret=interpret,
    )(tile_expert, n_tiles, xs, *([W1] * _NSPLIT),
      b1.reshape(E, 1, 4 * D), *([W2] * _NSPLIT), b2.reshape(E, 1, D))


_C_ROWS = T // NW
_LANES = 16


def _combine_sc(ys, posA, posB, wA, wB):
    @functools.partial(
        pl.kernel,
        mesh=plsc.VectorSubcoreMesh(**_SC_MESH),
        out_type=jax.ShapeDtypeStruct((T, D), jnp.float32),
        compiler_params=pltpu.CompilerParams(needs_layout_passes=False),
        scratch_types=[
            pltpu.VMEM((_C_ROWS,), jnp.int32),
            pltpu.VMEM((_C_ROWS,), jnp.int32),
            pltpu.VMEM((_C_ROWS,), jnp.float32),
            pltpu.VMEM((_C_ROWS,), jnp.float32),
            pltpu.VMEM((_C_ROWS, D), jnp.float32),
            pltpu.VMEM((_C_ROWS, D), jnp.float32),
            pltpu.SemaphoreType.DMA,
        ],
    )
    def k(ys_hbm, pa_hbm, pb_hbm, wa_hbm, wb_hbm, out_hbm,
          ia, ib, va, vb, ra, rb, sem):
        wid = lax.axis_index("s") * 2 + lax.axis_index("c")
        base = wid * _C_ROWS
        pltpu.sync_copy(pa_hbm.at[pl.ds(base, _C_ROWS)], ia)
        pltpu.sync_copy(pb_hbm.at[pl.ds(base, _C_ROWS)], ib)
        pltpu.sync_copy(wa_hbm.at[pl.ds(base, _C_ROWS)], va)
        pltpu.sync_copy(wb_hbm.at[pl.ds(base, _C_ROWS)], vb)
        ca = pltpu.async_copy(ys_hbm.at[ia], ra, sem)
        cb = pltpu.async_copy(ys_hbm.at[ib], rb, sem)
        ca.wait()
        cb.wait()

        def body(r, carry):
            ridx = jnp.full((_LANES,), r, jnp.int32)
            a = plsc.load_gather(va, [ridx])
            b = plsc.load_gather(vb, [ridx])
            for j in range(D // _LANES):
                s = pl.ds(j * _LANES, _LANES)
                ra[r, s] = a * ra[r, s] + b * rb[r, s]
            return carry

        lax.fori_loop(0, _C_ROWS, body, 0)
        pltpu.sync_copy(ra, out_hbm.at[pl.ds(base, _C_ROWS)])

    return k(ys, posA, posB, wA, wB)


def kernel(x, Wr, br, W1, b1, W2, b2):
    wa, wb, da, db, te, nt = _route_plan(x, Wr, br)
    posA = da.reshape(T)
    posB = db.reshape(T)
    xs = _scatter_sc(x, posA, posB)
    ys = _mlp(te.reshape(MAX_TILES), nt.reshape(1), xs, W1, b1, W2, b2)
    return _combine_sc(ys, posA, posB, wa.reshape(T), wb.reshape(T))


# speedup vs baseline: 1.2130x; 1.2130x over previous
import functools

import jax
import jax.numpy as jnp
from jax import lax
from jax.experimental import pallas as pl
from jax.experimental.pallas import tpu as pltpu
from jax.experimental.pallas import tpu_sc as plsc

D = 768
E = 64
T = 2048
TOPK = 2
TR = 128
MAX_TILES = T * TOPK // TR + E
T_PAD = MAX_TILES * TR
NW = 32
_BLK = 128


def _route_plan_body(x_ref, wr_ref, br_ref,
                     wa_ref, wb_ref, da_ref, db_ref, te_ref, nt_ref):
    f32 = jnp.float32
    logits = jnp.dot(x_ref[...], wr_ref[...], preferred_element_type=f32)
    logits = logits + br_ref[...]
    iota = lax.broadcasted_iota(jnp.int32, (T, E), 1)
    m1 = jnp.max(logits, axis=-1, keepdims=True)
    i1 = jnp.min(jnp.where(logits == m1, iota, E), axis=-1, keepdims=True)
    masked = jnp.where(iota == i1, -jnp.inf, logits)
    m2 = jnp.max(masked, axis=-1, keepdims=True)
    i2 = jnp.min(jnp.where(masked == m2, iota, E), axis=-1, keepdims=True)
    e2 = jnp.exp(m2 - m1)
    s = 1.0 + e2
    wa_ref[...] = 1.0 / s
    wb_ref[...] = e2 / s

    ohA = (iota == i1).astype(f32)
    ohB = (iota == i2).astype(f32)
    counts = jnp.sum(ohA + ohB, axis=0, keepdims=True)
    tiles = jnp.floor((counts + (TR - 1)) * (1.0 / TR))
    le = (lax.broadcasted_iota(jnp.int32, (E, E), 0)
          <= lax.broadcasted_iota(jnp.int32, (E, E), 1)).astype(f32)
    cum_tiles = jnp.dot(tiles, le, preferred_element_type=f32)
    padded_off = (cum_tiles - tiles) * TR
    nt = jnp.max(cum_tiles, axis=-1, keepdims=True)
    jcol = lax.broadcasted_iota(jnp.int32, (MAX_TILES, E), 0).astype(f32)
    jcl = jnp.minimum(jcol, nt - 1.0)
    te = jnp.sum(jnp.where(cum_tiles <= jcl, 1.0, 0.0), axis=-1, keepdims=True)
    te_ref[...] = te.astype(jnp.int32)
    nt_ref[...] = nt.astype(jnp.int32)

    bi = lax.broadcasted_iota(jnp.int32, (_BLK, _BLK), 0)
    bj = lax.broadcasted_iota(jnp.int32, (_BLK, _BLK), 1)
    lx = (bj < bi).astype(f32)
    carry = jnp.zeros((1, E), f32)
    for blk in range(T // _BLK):
        lo, hi = blk * _BLK, (blk + 1) * _BLK
        oa = ohA[lo:hi, :]
        ob = ohB[lo:hi, :]
        ra = jnp.dot(lx, oa, preferred_element_type=f32) + carry
        carry = carry + jnp.sum(oa, axis=0, keepdims=True)
        rb = jnp.dot(lx, ob, preferred_element_type=f32) + carry
        carry = carry + jnp.sum(ob, axis=0, keepdims=True)
        da = jnp.sum((ra + padded_off) * oa, axis=-1, keepdims=True)
        db = jnp.sum((rb + padded_off) * ob, axis=-1, keepdims=True)
        da_ref[lo:hi, :] = da.astype(jnp.int32)
        db_ref[lo:hi, :] = db.astype(jnp.int32)


def _route_plan(x, Wr, br, interpret=False):
    return pl.pallas_call(
        _route_plan_body,
        out_shape=[
            jax.ShapeDtypeStruct((T, 1), jnp.float32),
            jax.ShapeDtypeStruct((T, 1), jnp.float32),
            jax.ShapeDtypeStruct((T, 1), jnp.int32),
            jax.ShapeDtypeStruct((T, 1), jnp.int32),
            jax.ShapeDtypeStruct((MAX_TILES, 1), jnp.int32),
            jax.ShapeDtypeStruct((1, 1), jnp.int32),
        ],
        interpret=interpret,
    )(x, Wr, br.reshape(1, E))


_SC_MESH = dict(core_axis_name="c", subcore_axis_name="s")
_S_ROWS = T // NW


def _scatter_sc(x, da, db):
    @functools.partial(
        pl.kernel,
        mesh=plsc.VectorSubcoreMesh(**_SC_MESH),
        out_type=jax.ShapeDtypeStruct((T_PAD, D), jnp.float32),
        scratch_types=[
            pltpu.VMEM((_S_ROWS,), jnp.int32),
            pltpu.VMEM((_S_ROWS,), jnp.int32),
            pltpu.VMEM((_S_ROWS, D), jnp.float32),
            pltpu.SemaphoreType.DMA,
            pltpu.SemaphoreType.DMA,
        ],
    )
    def k(x_hbm, da_hbm, db_hbm, xs_hbm, ia, ib, rows_v, s1, s2):
        wid = lax.axis_index("s") * 2 + lax.axis_index("c")
        base = wid * _S_ROWS
        pltpu.sync_copy(x_hbm.at[pl.ds(base, _S_ROWS)], rows_v)
        pltpu.sync_copy(da_hbm.at[pl.ds(base, _S_ROWS)], ia)
        pltpu.sync_copy(db_hbm.at[pl.ds(base, _S_ROWS)], ib)
        ca = pltpu.async_copy(rows_v, xs_hbm.at[ia], s1)
        cb = pltpu.async_copy(rows_v, xs_hbm.at[ib], s2)
        ca.wait()
        cb.wait()

    return k(x, da, db)


_NSPLIT = 2


def _mlp_body(te_ref, nt_ref, xs_ref, *refs):
    w1_refs = refs[:_NSPLIT]
    b1_ref = refs[_NSPLIT]
    w2_refs = refs[_NSPLIT + 1:2 * _NSPLIT + 1]
    b2_ref = refs[2 * _NSPLIT + 1]
    ys_ref = refs[2 * _NSPLIT + 2]
    j = pl.program_id(0)
    hc = 4 * D // _NSPLIT

    @pl.when(j < nt_ref[0])
    def _():
        xt = xs_ref[...]
        y = b2_ref[0, 0, :]
        for c in range(_NSPLIT):
            h = jnp.dot(xt, w1_refs[c][0], preferred_element_type=jnp.float32)
            h = jnp.maximum(h + b1_ref[0, 0, c * hc:(c + 1) * hc], 0.0)
            y = y + jnp.dot(h, w2_refs[c][0],
                            preferred_element_type=jnp.float32)
        ys_ref[...] = y


def _mlp(tile_expert, n_tiles, xs, W1, b1, W2, b2, interpret=False):
    hc = 4 * D // _NSPLIT
    grid_spec = pltpu.PrefetchScalarGridSpec(
        num_scalar_prefetch=2,
        grid=(MAX_TILES,),
        in_specs=(
            [pl.BlockSpec((TR, D), lambda j, te, nt: (j, 0))]
            + [pl.BlockSpec((1, D, hc),
                            lambda j, te, nt, c=c: (te[j], 0, c))
               for c in range(_NSPLIT)]
            + [pl.BlockSpec((1, 1, 4 * D), lambda j, te, nt: (te[j], 0, 0))]
            + [pl.BlockSpec((1, hc, D),
                            lambda j, te, nt, c=c: (te[j], c, 0))
               for c in range(_NSPLIT)]
            + [pl.BlockSpec((1, 1, D), lambda j, te, nt: (te[j], 0, 0))]
        ),
        out_specs=pl.BlockSpec((TR, D), lambda j, te, nt: (j, 0)),
    )
    return pl.pallas_call(
        _mlp_body,
        grid_spec=grid_spec,
        out_shape=jax.ShapeDtypeStruct((T_PAD, D), jnp.float32),
        compiler_params=pltpu.CompilerParams(
            dimension_semantics=("arbitrary",),
            vmem_limit_bytes=120 * 1024 * 1024,
        ),
        interpret=interpret,
    )(tile_expert, n_tiles, xs, *([W1] * _NSPLIT),
      b1.reshape(E, 1, 4 * D), *([W2] * _NSPLIT), b2.reshape(E, 1, D))


_C_ROWS = T // NW
_LANES = 16


def _combine_sc(ys, posA, posB, wA, wB):
    @functools.partial(
        pl.kernel,
        mesh=plsc.VectorSubcoreMesh(**_SC_MESH),
        out_type=jax.ShapeDtypeStruct((T, D), jnp.float32),
        compiler_params=pltpu.CompilerParams(needs_layout_passes=False),
        scratch_types=[
            pltpu.VMEM((_C_ROWS,), jnp.int32),
            pltpu.VMEM((_C_ROWS,), jnp.int32),
            pltpu.VMEM((_C_ROWS,), jnp.float32),
            pltpu.VMEM((_C_ROWS,), jnp.float32),
            pltpu.VMEM((_C_ROWS, D), jnp.float32),
            pltpu.VMEM((_C_ROWS, D), jnp.float32),
            pltpu.SemaphoreType.DMA,
        ],
    )
    def k(ys_hbm, pa_hbm, pb_hbm, wa_hbm, wb_hbm, out_hbm,
          ia, ib, va, vb, ra, rb, sem):
        wid = lax.axis_index("s") * 2 + lax.axis_index("c")
        base = wid * _C_ROWS
        pltpu.sync_copy(pa_hbm.at[pl.ds(base, _C_ROWS)], ia)
        pltpu.sync_copy(pb_hbm.at[pl.ds(base, _C_ROWS)], ib)
        pltpu.sync_copy(wa_hbm.at[pl.ds(base, _C_ROWS)], va)
        pltpu.sync_copy(wb_hbm.at[pl.ds(base, _C_ROWS)], vb)
        ca = pltpu.async_copy(ys_hbm.at[ia], ra, sem)
        cb = pltpu.async_copy(ys_hbm.at[ib], rb, sem)
        ca.wait()
        cb.wait()

        def body(r, carry):
            ridx = jnp.full((_LANES,), r, jnp.int32)
            a = plsc.load_gather(va, [ridx])
            b = plsc.load_gather(vb, [ridx])
            for j in range(D // _LANES):
                s = pl.ds(j * _LANES, _LANES)
                ra[r, s] = a * ra[r, s] + b * rb[r, s]
            return carry

        lax.fori_loop(0, _C_ROWS, body, 0)
        pltpu.sync_copy(ra, out_hbm.at[pl.ds(base, _C_ROWS)])

    return k(ys, posA, posB, wA, wB)


def kernel(x, Wr, br, W1, b1, W2, b2):
    wa, wb, da, db, te, nt = _route_plan(x, Wr, br)
    posA = da.reshape(T)
    posB = db.reshape(T)
    xs = _scatter_sc(x, posA, posB)
    ys = _mlp(te.reshape(MAX_TILES), nt.reshape(1), xs, W1, b1, W2, b2)
    return _combine_sc(ys, posA, posB, wa.reshape(T), wb.reshape(T))
